# Initial kernel scaffold; baseline (speedup 1.0000x reference)
import jax
import jax.numpy as jnp
from jax.experimental import pallas as pl


def kernel(user_emb, item_emb, norm_idx, norm_val, sym_idx, sym_val, herb_idx, herb_val, Q_user, W_gc_user, b_gc_user, Q_item, W_gc_item, b_gc_item, M_user, M_item, W_mlp_user, b_mlp_user):
    raise NotImplementedError("write your pallas kernel here")


# speedup vs baseline: 2.7948x; 2.7948x over previous
import functools

import jax
import jax.numpy as jnp
from jax import lax
from jax.experimental import pallas as pl
from jax.experimental.pallas import tpu as pltpu
from jax.experimental.pallas import tpu_sc as plsc

NU = 25000
NI = 25000
NN = NU + NI
D = 64
NC = 2
NS = 16
H = 25000
HP = 25088
ROWS_T = HP // NS
CH = 128
INNER = 14
SUP = CH * INNER
NSUP_A = 28
NSUP_B = 7
EA = NS * NSUP_A * SUP
EB = NC * NS * NSUP_B * SUP
ZR = 56


def _sc_body(pre, nrow, ncol, nval, srow, scol, sval, hrow, hcol, hval,
             e_out, tu_out, ti_out,
             rowb, colb, valb, idx_g, idx_s, vsc, gbuf, zbuf, acc, sem):
    cid = lax.axis_index("c")
    sid = lax.axis_index("s")

    def zrow(r, carry):
        for j in range(D // 16):
            zbuf[r, pl.ds(j * 16, 16)] = jnp.zeros((16,), jnp.float32)
        return carry

    lax.fori_loop(0, ZR, zrow, 0)

    def run_phase(row_h, col_h, val_h, n_super, split_rows, col_off, out_h):
        zbase = sid * ROWS_T
        for z in range(ROWS_T // ZR):
            pltpu.sync_copy(zbuf, acc.at[pl.ds(zbase + z * ZR, ZR)])
        plsc.subcore_barrier()

        if split_rows:
            edge_base = sid * (n_super * SUP)
        else:
            edge_base = (cid * NS + sid) * (n_super * SUP)

        def super_body(g, carry):
            sb = edge_base + g * SUP
            pltpu.sync_copy(row_h.at[pl.ds(sb, SUP)], rowb)
            pltpu.sync_copy(col_h.at[pl.ds(sb, SUP)], colb)
            pltpu.sync_copy(val_h.at[pl.ds(sb, SUP)], valb)

            def chunk_body(ci, carry2):
                cb = ci * CH
                for j in range(CH // 16):
                    sl16 = pl.ds(j * 16, 16)
                    slb = pl.ds(cb + j * 16, 16)
                    r = rowb[slb]
                    cc = colb[slb]
                    vv = valb[slb]
                    idx_g[sl16] = cc + col_off
                    if split_rows:
                        idx_s[sl16] = r - jnp.where(r >= H, H, 0)
                        lo = cid * H
                        ok = (r >= lo) & (r < lo + H)
                        vsc[sl16] = jnp.where(ok, vv, 0.0)
                    else:
                        idx_s[sl16] = r
                        vsc[sl16] = vv
                pltpu.async_copy(pre.at[idx_g], gbuf, sem).wait()

                def scale_body(g, carry3):
                    v16 = vsc[pl.ds(g * 16, 16)]
                    base = g * 16
                    for e in range(16):
                        v = v16[e]
                        for j in range(D // 16):
                            sl = pl.ds(j * 16, 16)
                            gbuf[base + e, sl] = gbuf[base + e, sl] * v
                    return carry3

                lax.fori_loop(0, CH // 16, scale_body, 0)
                pltpu.sync_copy(gbuf, acc.at[idx_s], add=True)
                return carry2

            lax.fori_loop(0, INNER, chunk_body, 0)
            return carry

        lax.fori_loop(0, n_super, super_body, 0)
        plsc.subcore_barrier()
        pltpu.sync_copy(acc.at[pl.ds(sid * ROWS_T, ROWS_T)],
                        out_h.at[cid, pl.ds(sid * ROWS_T, ROWS_T)])
        plsc.subcore_barrier()

    run_phase(nrow, ncol, nval, NSUP_A, True, 0, e_out)
    run_phase(srow, scol, sval, NSUP_B, False, 0, tu_out)
    run_phase(hrow, hcol, hval, NSUP_B, False, NU, ti_out)


_sc_spmm = functools.partial(
    pl.kernel,
    out_type=[
        jax.ShapeDtypeStruct((NC, HP, D), jnp.float32),
        jax.ShapeDtypeStruct((NC, HP, D), jnp.float32),
        jax.ShapeDtypeStruct((NC, HP, D), jnp.float32),
    ],
    mesh=plsc.VectorSubcoreMesh(
        core_axis_name="c", subcore_axis_name="s",
        num_cores=NC, num_subcores=NS),
    compiler_params=pltpu.CompilerParams(use_tc_tiling_on_sc=False),
    scratch_types=[
        pltpu.VMEM((SUP,), jnp.int32),
        pltpu.VMEM((SUP,), jnp.int32),
        pltpu.VMEM((SUP,), jnp.float32),
        pltpu.VMEM((CH,), jnp.int32),
        pltpu.VMEM((CH,), jnp.int32),
        pltpu.VMEM((CH,), jnp.float32),
        pltpu.VMEM((CH, D), jnp.float32),
        pltpu.VMEM((ZR, D), jnp.float32),
        pltpu.VMEM_SHARED((HP, D), jnp.float32),
        pltpu.SemaphoreType.DMA,
    ],
)(_sc_body)


def _pad_edges(idx, val, total, mod):
    e = val.shape[0]
    p = total - e
    ar = jnp.arange(p, dtype=jnp.int32)
    fill = (ar * 7) % mod
    row = jnp.concatenate([idx[0], fill])
    col = jnp.concatenate([idx[1], fill])
    valp = jnp.concatenate([val, jnp.zeros((p,), val.dtype)])
    return row, col, valp


def _user_body(e_ref, u_ref, t0_ref, t1_ref, q_ref, w1_ref, w2_ref, b_ref,
               m_ref, wm_ref, bm_ref, o_ref):
    t = jnp.tanh(e_ref[0] @ q_ref[...])
    g = jnp.tanh(u_ref[...] @ w1_ref[...] + t @ w2_ref[...] + b_ref[...])
    n = jnp.sqrt(jnp.sum(g * g, axis=1, keepdims=True))
    g = g / jnp.maximum(n, 1e-12)
    pair = jnp.tanh((t0_ref[0] + t1_ref[0]) @ m_ref[...])
    ug = g + pair
    o_ref[...] = jnp.tanh(ug @ wm_ref[...] + bm_ref[...])


def _item_body(e_ref, i_ref, t0_ref, t1_ref, q_ref, w1_ref, w2_ref, b_ref,
               m_ref, o_ref):
    t = jnp.tanh(e_ref[0] @ q_ref[...])
    g = jnp.tanh(i_ref[...] @ w1_ref[...] + t @ w2_ref[...] + b_ref[...])
    n = jnp.sqrt(jnp.sum(g * g, axis=1, keepdims=True))
    g = g / jnp.maximum(n, 1e-12)
    o_ref[...] = g + jnp.tanh((t0_ref[0] + t1_ref[0]) @ m_ref[...])


_BT = 1000


def _row_spec(half):
    return pl.BlockSpec((1, _BT, D), lambda i, h=half: (h, i, 0))


def _full(shape):
    return pl.BlockSpec(shape, lambda i: tuple(0 for _ in shape))


def kernel(user_emb, item_emb, norm_idx, norm_val, sym_idx, sym_val,
           herb_idx, herb_val, Q_user, W_gc_user, b_gc_user, Q_item,
           W_gc_item, b_gc_item, M_user, M_item, W_mlp_user, b_mlp_user):
    pre = jnp.concatenate([user_emb, item_emb], axis=0)
    nrow, ncol, nval = _pad_edges(norm_idx, norm_val, EA, NN)
    srow, scol, sval = _pad_edges(sym_idx, sym_val, EB, NU)
    hrow, hcol, hval = _pad_edges(herb_idx, herb_val, EB, NI)

    e2, tup, tip = _sc_spmm(pre, nrow, ncol, nval, srow, scol, sval,
                            hrow, hcol, hval)

    w1u, w2u = W_gc_user[:D], W_gc_user[D:]
    w1i, w2i = W_gc_item[:D], W_gc_item[D:]
    grid = (NU // _BT,)

    out_u = pl.pallas_call(
        _user_body,
        grid=grid,
        in_specs=[
            _row_spec(0),
            pl.BlockSpec((_BT, D), lambda i: (i, 0)),
            _row_spec(0), _row_spec(1),
            _full((D, D)), _full((D, D)), _full((D, D)), _full((1, D)),
            _full((D, D)), _full((D, D)), _full((1, D)),
        ],
        out_specs=pl.BlockSpec((_BT, D), lambda i: (i, 0)),
        out_shape=jax.ShapeDtypeStruct((NU, D), jnp.float32),
    )(e2, user_emb, tup, tup, Q_user, w1u, w2u, b_gc_user,
      M_user, W_mlp_user, b_mlp_user)

    out_i = pl.pallas_call(
        _item_body,
        grid=grid,
        in_specs=[
            _row_spec(1),
            pl.BlockSpec((_BT, D), lambda i: (i, 0)),
            _row_spec(0), _row_spec(1),
            _full((D, D)), _full((D, D)), _full((D, D)), _full((1, D)),
            _full((D, D)),
        ],
        out_specs=pl.BlockSpec((_BT, D), lambda i: (i, 0)),
        out_shape=jax.ShapeDtypeStruct((NI, D), jnp.float32),
    )(e2, item_emb, tip, tip, Q_item, w1i, w2i, b_gc_item, M_item)

    return jnp.concatenate([out_u, out_i], axis=0)


# speedup vs baseline: 3.5207x; 1.2597x over previous
import functools

import jax
import jax.numpy as jnp
from jax import lax
from jax.experimental import pallas as pl
from jax.experimental.pallas import tpu as pltpu
from jax.experimental.pallas import tpu_sc as plsc

NU = 25000
NI = 25000
NN = NU + NI
D = 64
NC = 2
NS = 16
H = 25000
HP = 25088
ROWS_T = HP // NS
CH = 128
INNER = 14
SUP = CH * INNER
NSUP_A = 28
NSUP_B = 7
EA = NS * NSUP_A * SUP
EB = NC * NS * NSUP_B * SUP
ZR = 56


def _sc_body(pre, nrow, ncol, nval, srow, scol, sval, hrow, hcol, hval,
             e_out, tu_out, ti_out,
             rowb, colb, valb, idx_g0, idx_s0, vsc0, idx_g1, idx_s1, vsc1,
             gb0, gb1, zbuf, acc, sem0, sem1):
    cid = lax.axis_index("c")
    sid = lax.axis_index("s")

    def zrow(r, carry):
        for j in range(D // 16):
            zbuf[r, pl.ds(j * 16, 16)] = jnp.zeros((16,), jnp.float32)
        return carry

    lax.fori_loop(0, ZR, zrow, 0)

    def run_phase(row_h, col_h, val_h, n_super, split_rows, col_off, out_h):
        zbase = sid * ROWS_T
        for z in range(ROWS_T // ZR):
            pltpu.sync_copy(zbuf, acc.at[pl.ds(zbase + z * ZR, ZR)])
        plsc.subcore_barrier()

        if split_rows:
            edge_base = sid * (n_super * SUP)
        else:
            edge_base = (cid * NS + sid) * (n_super * SUP)

        def compute_idx(ci, ig, isc, vs):
            cb = ci * CH
            for j in range(CH // 16):
                sl16 = pl.ds(j * 16, 16)
                slb = pl.ds(cb + j * 16, 16)
                r = rowb[slb]
                cc = colb[slb]
                vv = valb[slb]
                ig[sl16] = cc + col_off
                if split_rows:
                    isc[sl16] = r - jnp.where(r >= H, H, 0)
                    lo = cid * H
                    ok = (r >= lo) & (r < lo + H)
                    vs[sl16] = jnp.where(ok, vv, 0.0)
                else:
                    isc[sl16] = r
                    vs[sl16] = vv

        def scale_scatter(gb, vs, isc):
            def scale_body(g, carry3):
                v16 = vs[pl.ds(g * 16, 16)]
                base = g * 16
                for e in range(16):
                    v = v16[e]
                    for j in range(D // 16):
                        sl = pl.ds(j * 16, 16)
                        gb[base + e, sl] = gb[base + e, sl] * v
                return carry3

            lax.fori_loop(0, CH // 16, scale_body, 0)
            pltpu.sync_copy(gb, acc.at[isc], add=True)

        def super_body(g, carry):
            sb = edge_base + g * SUP
            pltpu.sync_copy(row_h.at[pl.ds(sb, SUP)], rowb)
            pltpu.sync_copy(col_h.at[pl.ds(sb, SUP)], colb)
            pltpu.sync_copy(val_h.at[pl.ds(sb, SUP)], valb)

            compute_idx(0, idx_g0, idx_s0, vsc0)
            pltpu.async_copy(pre.at[idx_g0], gb0, sem0)

            def pair_body(h, carry2):
                compute_idx(2 * h + 1, idx_g1, idx_s1, vsc1)
                pltpu.async_copy(pre.at[idx_g1], gb1, sem1)
                pltpu.make_async_copy(pre.at[idx_g0], gb0, sem0).wait()
                scale_scatter(gb0, vsc0, idx_s0)

                @pl.when(h < INNER // 2 - 1)
                def _():
                    compute_idx(2 * h + 2, idx_g0, idx_s0, vsc0)
                    pltpu.async_copy(pre.at[idx_g0], gb0, sem0)

                pltpu.make_async_copy(pre.at[idx_g1], gb1, sem1).wait()
                scale_scatter(gb1, vsc1, idx_s1)
                return carry2

            lax.fori_loop(0, INNER // 2, pair_body, 0)
            return carry

        lax.fori_loop(0, n_super, super_body, 0)
        plsc.subcore_barrier()
        pltpu.sync_copy(acc.at[pl.ds(sid * ROWS_T, ROWS_T)],
                        out_h.at[cid, pl.ds(sid * ROWS_T, ROWS_T)])
        plsc.subcore_barrier()

    run_phase(nrow, ncol, nval, NSUP_A, True, 0, e_out)
    run_phase(srow, scol, sval, NSUP_B, False, 0, tu_out)
    run_phase(hrow, hcol, hval, NSUP_B, False, NU, ti_out)


_sc_spmm = functools.partial(
    pl.kernel,
    out_type=[
        jax.ShapeDtypeStruct((NC, HP, D), jnp.float32),
        jax.ShapeDtypeStruct((NC, HP, D), jnp.float32),
        jax.ShapeDtypeStruct((NC, HP, D), jnp.float32),
    ],
    mesh=plsc.VectorSubcoreMesh(
        core_axis_name="c", subcore_axis_name="s",
        num_cores=NC, num_subcores=NS),
    compiler_params=pltpu.CompilerParams(use_tc_tiling_on_sc=False),
    scratch_types=[
        pltpu.VMEM((SUP,), jnp.int32),
        pltpu.VMEM((SUP,), jnp.int32),
        pltpu.VMEM((SUP,), jnp.float32),
        pltpu.VMEM((CH,), jnp.int32),
        pltpu.VMEM((CH,), jnp.int32),
        pltpu.VMEM((CH,), jnp.float32),
        pltpu.VMEM((CH,), jnp.int32),
        pltpu.VMEM((CH,), jnp.int32),
        pltpu.VMEM((CH,), jnp.float32),
        pltpu.VMEM((CH, D), jnp.float32),
        pltpu.VMEM((CH, D), jnp.float32),
        pltpu.VMEM((ZR, D), jnp.float32),
        pltpu.VMEM_SHARED((HP, D), jnp.float32),
        pltpu.SemaphoreType.DMA,
        pltpu.SemaphoreType.DMA,
    ],
)(_sc_body)


def _pad_edges(idx, val, total, mod):
    e = val.shape[0]
    p = total - e
    ar = jnp.arange(p, dtype=jnp.int32)
    fill = (ar * 7) % mod
    row = jnp.concatenate([idx[0], fill])
    col = jnp.concatenate([idx[1], fill])
    valp = jnp.concatenate([val, jnp.zeros((p,), val.dtype)])
    return row, col, valp


def _user_body(e_ref, u_ref, t0_ref, t1_ref, q_ref, w1_ref, w2_ref, b_ref,
               m_ref, wm_ref, bm_ref, o_ref):
    t = jnp.tanh(e_ref[0] @ q_ref[...])
    g = jnp.tanh(u_ref[...] @ w1_ref[...] + t @ w2_ref[...] + b_ref[...])
    n = jnp.sqrt(jnp.sum(g * g, axis=1, keepdims=True))
    g = g / jnp.maximum(n, 1e-12)
    pair = jnp.tanh((t0_ref[0] + t1_ref[0]) @ m_ref[...])
    ug = g + pair
    o_ref[...] = jnp.tanh(ug @ wm_ref[...] + bm_ref[...])


def _item_body(e_ref, i_ref, t0_ref, t1_ref, q_ref, w1_ref, w2_ref, b_ref,
               m_ref, o_ref):
    t = jnp.tanh(e_ref[0] @ q_ref[...])
    g = jnp.tanh(i_ref[...] @ w1_ref[...] + t @ w2_ref[...] + b_ref[...])
    n = jnp.sqrt(jnp.sum(g * g, axis=1, keepdims=True))
    g = g / jnp.maximum(n, 1e-12)
    o_ref[...] = g + jnp.tanh((t0_ref[0] + t1_ref[0]) @ m_ref[...])


_BT = 1000


def _row_spec(half):
    return pl.BlockSpec((1, _BT, D), lambda i, h=half: (h, i, 0))


def _full(shape):
    return pl.BlockSpec(shape, lambda i: tuple(0 for _ in shape))


def kernel(user_emb, item_emb, norm_idx, norm_val, sym_idx, sym_val,
           herb_idx, herb_val, Q_user, W_gc_user, b_gc_user, Q_item,
           W_gc_item, b_gc_item, M_user, M_item, W_mlp_user, b_mlp_user):
    pre = jnp.concatenate([user_emb, item_emb], axis=0)
    nrow, ncol, nval = _pad_edges(norm_idx, norm_val, EA, NN)
    srow, scol, sval = _pad_edges(sym_idx, sym_val, EB, NU)
    hrow, hcol, hval = _pad_edges(herb_idx, herb_val, EB, NI)

    e2, tup, tip = _sc_spmm(pre, nrow, ncol, nval, srow, scol, sval,
                            hrow, hcol, hval)

    w1u, w2u = W_gc_user[:D], W_gc_user[D:]
    w1i, w2i = W_gc_item[:D], W_gc_item[D:]
    grid = (NU // _BT,)

    out_u = pl.pallas_call(
        _user_body,
        grid=grid,
        in_specs=[
            _row_spec(0),
            pl.BlockSpec((_BT, D), lambda i: (i, 0)),
            _row_spec(0), _row_spec(1),
            _full((D, D)), _full((D, D)), _full((D, D)), _full((1, D)),
            _full((D, D)), _full((D, D)), _full((1, D)),
        ],
        out_specs=pl.BlockSpec((_BT, D), lambda i: (i, 0)),
        out_shape=jax.ShapeDtypeStruct((NU, D), jnp.float32),
    )(e2, user_emb, tup, tup, Q_user, w1u, w2u, b_gc_user,
      M_user, W_mlp_user, b_mlp_user)

    out_i = pl.pallas_call(
        _item_body,
        grid=grid,
        in_specs=[
            _row_spec(1),
            pl.BlockSpec((_BT, D), lambda i: (i, 0)),
            _row_spec(0), _row_spec(1),
            _full((D, D)), _full((D, D)), _full((D, D)), _full((1, D)),
            _full((D, D)),
        ],
        out_specs=pl.BlockSpec((_BT, D), lambda i: (i, 0)),
        out_shape=jax.ShapeDtypeStruct((NI, D), jnp.float32),
    )(e2, item_emb, tip, tip, Q_item, w1i, w2i, b_gc_item, M_item)

    return jnp.concatenate([out_u, out_i], axis=0)


# speedup vs baseline: 3.5462x; 1.0073x over previous
import functools

import jax
import jax.numpy as jnp
from jax import lax
from jax.experimental import pallas as pl
from jax.experimental.pallas import tpu as pltpu
from jax.experimental.pallas import tpu_sc as plsc

NU = 25000
NI = 25000
NN = NU + NI
D = 64
NC = 2
NS = 16
H = 25000
HP = 25088
ROWS_T = HP // NS
CH = 128
INNER = 14
SUP = CH * INNER
NSUP_A = 28
NSUP_B = 7
EA = NS * NSUP_A * SUP
EB = NC * NS * NSUP_B * SUP
ZR = 56


def _sc_body(pre, nrow, ncol, nval, srow, scol, sval, hrow, hcol, hval,
             e_out, tu_out, ti_out,
             rowb, colb, valb, idx_g0, idx_s0, vsc0, idx_g1, idx_s1, vsc1,
             gb0, gb1, zbuf, acc, sem0, sem1, sem_s0, sem_s1):
    cid = lax.axis_index("c")
    sid = lax.axis_index("s")

    def zrow(r, carry):
        for j in range(D // 16):
            zbuf[r, pl.ds(j * 16, 16)] = jnp.zeros((16,), jnp.float32)
        return carry

    lax.fori_loop(0, ZR, zrow, 0)

    def run_phase(row_h, col_h, val_h, n_super, split_rows, col_off, out_h):
        zbase = sid * ROWS_T
        for z in range(ROWS_T // ZR):
            pltpu.sync_copy(zbuf, acc.at[pl.ds(zbase + z * ZR, ZR)])
        plsc.subcore_barrier()

        if split_rows:
            edge_base = sid * (n_super * SUP)
        else:
            edge_base = (cid * NS + sid) * (n_super * SUP)

        def compute_idx(ci, ig, isc, vs):
            cb = ci * CH
            for j in range(CH // 16):
                sl16 = pl.ds(j * 16, 16)
                slb = pl.ds(cb + j * 16, 16)
                r = rowb[slb]
                cc = colb[slb]
                vv = valb[slb]
                ig[sl16] = cc + col_off
                if split_rows:
                    isc[sl16] = r - jnp.where(r >= H, H, 0)
                    lo = cid * H
                    ok = (r >= lo) & (r < lo + H)
                    vs[sl16] = jnp.where(ok, vv, 0.0)
                else:
                    isc[sl16] = r
                    vs[sl16] = vv

        def scale_scatter(gb, vs, isc, sem_s):
            def scale_body(g, carry3):
                v16 = vs[pl.ds(g * 16, 16)]
                base = g * 16
                for e in range(16):
                    v = v16[e]
                    for j in range(D // 16):
                        sl = pl.ds(j * 16, 16)
                        gb[base + e, sl] = gb[base + e, sl] * v
                return carry3

            lax.fori_loop(0, CH // 16, scale_body, 0)
            pltpu.async_copy(gb, acc.at[isc], sem_s, add=True)

        def wait_scatter(gb, isc, sem_s):
            pltpu.make_async_copy(gb, acc.at[isc], sem_s).wait()

        def super_body(g, carry):
            sb = edge_base + g * SUP
            pltpu.sync_copy(row_h.at[pl.ds(sb, SUP)], rowb)
            pltpu.sync_copy(col_h.at[pl.ds(sb, SUP)], colb)
            pltpu.sync_copy(val_h.at[pl.ds(sb, SUP)], valb)

            compute_idx(0, idx_g0, idx_s0, vsc0)
            pltpu.async_copy(pre.at[idx_g0], gb0, sem0)

            def pair_body(h, carry2):
                @pl.when(h > 0)
                def _():
                    wait_scatter(gb1, idx_s1, sem_s1)

                compute_idx(2 * h + 1, idx_g1, idx_s1, vsc1)
                pltpu.async_copy(pre.at[idx_g1], gb1, sem1)
                pltpu.make_async_copy(pre.at[idx_g0], gb0, sem0).wait()
                scale_scatter(gb0, vsc0, idx_s0, sem_s0)

                @pl.when(h < INNER // 2 - 1)
                def _():
                    wait_scatter(gb0, idx_s0, sem_s0)
                    compute_idx(2 * h + 2, idx_g0, idx_s0, vsc0)
                    pltpu.async_copy(pre.at[idx_g0], gb0, sem0)

                pltpu.make_async_copy(pre.at[idx_g1], gb1, sem1).wait()
                scale_scatter(gb1, vsc1, idx_s1, sem_s1)
                return carry2

            lax.fori_loop(0, INNER // 2, pair_body, 0)
            wait_scatter(gb0, idx_s0, sem_s0)
            wait_scatter(gb1, idx_s1, sem_s1)
            return carry

        lax.fori_loop(0, n_super, super_body, 0)
        plsc.subcore_barrier()
        pltpu.sync_copy(acc.at[pl.ds(sid * ROWS_T, ROWS_T)],
                        out_h.at[cid, pl.ds(sid * ROWS_T, ROWS_T)])
        plsc.subcore_barrier()

    run_phase(nrow, ncol, nval, NSUP_A, True, 0, e_out)
    run_phase(srow, scol, sval, NSUP_B, False, 0, tu_out)
    run_phase(hrow, hcol, hval, NSUP_B, False, NU, ti_out)


_sc_spmm = functools.partial(
    pl.kernel,
    out_type=[
        jax.ShapeDtypeStruct((NC, HP, D), jnp.float32),
        jax.ShapeDtypeStruct((NC, HP, D), jnp.float32),
        jax.ShapeDtypeStruct((NC, HP, D), jnp.float32),
    ],
    mesh=plsc.VectorSubcoreMesh(
        core_axis_name="c", subcore_axis_name="s",
        num_cores=NC, num_subcores=NS),
    compiler_params=pltpu.CompilerParams(use_tc_tiling_on_sc=False),
    scratch_types=[
        pltpu.VMEM((SUP,), jnp.int32),
        pltpu.VMEM((SUP,), jnp.int32),
        pltpu.VMEM((SUP,), jnp.float32),
        pltpu.VMEM((CH,), jnp.int32),
        pltpu.VMEM((CH,), jnp.int32),
        pltpu.VMEM((CH,), jnp.float32),
        pltpu.VMEM((CH,), jnp.int32),
        pltpu.VMEM((CH,), jnp.int32),
        pltpu.VMEM((CH,), jnp.float32),
        pltpu.VMEM((CH, D), jnp.float32),
        pltpu.VMEM((CH, D), jnp.float32),
        pltpu.VMEM((ZR, D), jnp.float32),
        pltpu.VMEM_SHARED((HP, D), jnp.float32),
        pltpu.SemaphoreType.DMA,
        pltpu.SemaphoreType.DMA,
        pltpu.SemaphoreType.DMA,
        pltpu.SemaphoreType.DMA,
    ],
)(_sc_body)


def _pad_edges(idx, val, total, mod):
    e = val.shape[0]
    p = total - e
    ar = jnp.arange(p, dtype=jnp.int32)
    fill = (ar * 7) % mod
    row = jnp.concatenate([idx[0], fill])
    col = jnp.concatenate([idx[1], fill])
    valp = jnp.concatenate([val, jnp.zeros((p,), val.dtype)])
    return row, col, valp


def _user_body(e_ref, u_ref, t0_ref, t1_ref, q_ref, w1_ref, w2_ref, b_ref,
               m_ref, wm_ref, bm_ref, o_ref):
    t = jnp.tanh(e_ref[0] @ q_ref[...])
    g = jnp.tanh(u_ref[...] @ w1_ref[...] + t @ w2_ref[...] + b_ref[...])
    n = jnp.sqrt(jnp.sum(g * g, axis=1, keepdims=True))
    g = g / jnp.maximum(n, 1e-12)
    pair = jnp.tanh((t0_ref[0] + t1_ref[0]) @ m_ref[...])
    ug = g + pair
    o_ref[...] = jnp.tanh(ug @ wm_ref[...] + bm_ref[...])


def _item_body(e_ref, i_ref, t0_ref, t1_ref, q_ref, w1_ref, w2_ref, b_ref,
               m_ref, o_ref):
    t = jnp.tanh(e_ref[0] @ q_ref[...])
    g = jnp.tanh(i_ref[...] @ w1_ref[...] + t @ w2_ref[...] + b_ref[...])
    n = jnp.sqrt(jnp.sum(g * g, axis=1, keepdims=True))
    g = g / jnp.maximum(n, 1e-12)
    o_ref[...] = g + jnp.tanh((t0_ref[0] + t1_ref[0]) @ m_ref[...])


_BT = 1000


def _row_spec(half):
    return pl.BlockSpec((1, _BT, D), lambda i, h=half: (h, i, 0))


def _full(shape):
    return pl.BlockSpec(shape, lambda i: tuple(0 for _ in shape))


def kernel(user_emb, item_emb, norm_idx, norm_val, sym_idx, sym_val,
           herb_idx, herb_val, Q_user, W_gc_user, b_gc_user, Q_item,
           W_gc_item, b_gc_item, M_user, M_item, W_mlp_user, b_mlp_user):
    pre = jnp.concatenate([user_emb, item_emb], axis=0)
    nrow, ncol, nval = _pad_edges(norm_idx, norm_val, EA, NN)
    srow, scol, sval = _pad_edges(sym_idx, sym_val, EB, NU)
    hrow, hcol, hval = _pad_edges(herb_idx, herb_val, EB, NI)

    e2, tup, tip = _sc_spmm(pre, nrow, ncol, nval, srow, scol, sval,
                            hrow, hcol, hval)

    w1u, w2u = W_gc_user[:D], W_gc_user[D:]
    w1i, w2i = W_gc_item[:D], W_gc_item[D:]
    grid = (NU // _BT,)

    out_u = pl.pallas_call(
        _user_body,
        grid=grid,
        in_specs=[
            _row_spec(0),
            pl.BlockSpec((_BT, D), lambda i: (i, 0)),
            _row_spec(0), _row_spec(1),
            _full((D, D)), _full((D, D)), _full((D, D)), _full((1, D)),
            _full((D, D)), _full((D, D)), _full((1, D)),
        ],
        out_specs=pl.BlockSpec((_BT, D), lambda i: (i, 0)),
        out_shape=jax.ShapeDtypeStruct((NU, D), jnp.float32),
    )(e2, user_emb, tup, tup, Q_user, w1u, w2u, b_gc_user,
      M_user, W_mlp_user, b_mlp_user)

    out_i = pl.pallas_call(
        _item_body,
        grid=grid,
        in_specs=[
            _row_spec(1),
            pl.BlockSpec((_BT, D), lambda i: (i, 0)),
            _row_spec(0), _row_spec(1),
            _full((D, D)), _full((D, D)), _full((D, D)), _full((1, D)),
            _full((D, D)),
        ],
        out_specs=pl.BlockSpec((_BT, D), lambda i: (i, 0)),
        out_shape=jax.ShapeDtypeStruct((NI, D), jnp.float32),
    )(e2, item_emb, tip, tip, Q_item, w1i, w2i, b_gc_item, M_item)

    return jnp.concatenate([out_u, out_i], axis=0)


# speedup vs baseline: 6.2507x; 1.7626x over previous
import functools

import jax
import jax.numpy as jnp
from jax import lax
from jax.experimental import pallas as pl
from jax.experimental.pallas import tpu as pltpu
from jax.experimental.pallas import tpu_sc as plsc

NU = 25000
NI = 25000
NN = NU + NI
D = 64
NC = 2
NS = 16
H = 25000
HP = 25088
ROWS_T = HP // NS
CH = 128
INNER = 14
SUP = CH * INNER
NSUP_A = 28
NSUP_B = 7
EA = NS * NSUP_A * SUP
EB = NC * NS * NSUP_B * SUP
ZR = 56


def _sc_body(pre, nrow, ncol, nval, srow, scol, sval, hrow, hcol, hval,
             e_out, tu_out, ti_out,
             rowb, colb, valb, idx_g0, idx_s0, vsc0, idx_g1, idx_s1, vsc1,
             gb0, gb1, zbuf, acc, sem0, sem1, sem_s0, sem_s1):
    cid = lax.axis_index("c")
    sid = lax.axis_index("s")
    ii16 = lax.broadcasted_iota(jnp.int32, (16,), 0)
    lanes = [ii16 * 0 + e for e in range(16)]

    def zrow(r, carry):
        for j in range(D // 16):
            zbuf[r, pl.ds(j * 16, 16)] = jnp.zeros((16,), jnp.float32)
        return carry

    lax.fori_loop(0, ZR, zrow, 0)

    def run_phase(row_h, col_h, val_h, n_super, split_rows, col_off, out_h):
        zbase = sid * ROWS_T
        for z in range(ROWS_T // ZR):
            pltpu.sync_copy(zbuf, acc.at[pl.ds(zbase + z * ZR, ZR)])
        plsc.subcore_barrier()

        if split_rows:
            edge_base = sid * (n_super * SUP)
        else:
            edge_base = (cid * NS + sid) * (n_super * SUP)

        def compute_idx(ci, ig, isc, vs):
            cb = ci * CH
            for j in range(CH // 16):
                sl16 = pl.ds(j * 16, 16)
                slb = pl.ds(cb + j * 16, 16)
                r = rowb[slb]
                cc = colb[slb]
                vv = valb[slb]
                ig[sl16] = cc + col_off
                if split_rows:
                    isc[sl16] = r - jnp.where(r >= H, H, 0)
                    lo = cid * H
                    ok = (r >= lo) & (r < lo + H)
                    vs[sl16] = jnp.where(ok, vv, 0.0)
                else:
                    isc[sl16] = r
                    vs[sl16] = vv

        def scale_scatter(gb, vs, isc, sem_s):
            def scale_body(g, carry3):
                v16 = vs[pl.ds(g * 16, 16)]
                base = g * 16
                for e in range(16):
                    bc = v16.at[lanes[e]].get(mode="promise_in_bounds")
                    for j in range(D // 16):
                        sl = pl.ds(j * 16, 16)
                        gb[base + e, sl] = gb[base + e, sl] * bc
                return carry3

            lax.fori_loop(0, CH // 16, scale_body, 0)
            pltpu.async_copy(gb, acc.at[isc], sem_s, add=True)

        def wait_scatter(gb, isc, sem_s):
            pltpu.make_async_copy(gb, acc.at[isc], sem_s).wait()

        def super_body(g, carry):
            sb = edge_base + g * SUP
            pltpu.sync_copy(row_h.at[pl.ds(sb, SUP)], rowb)
            pltpu.sync_copy(col_h.at[pl.ds(sb, SUP)], colb)
            pltpu.sync_copy(val_h.at[pl.ds(sb, SUP)], valb)

            compute_idx(0, idx_g0, idx_s0, vsc0)
            pltpu.async_copy(pre.at[idx_g0], gb0, sem0)

            def pair_body(h, carry2):
                @pl.when(h > 0)
                def _():
                    wait_scatter(gb1, idx_s1, sem_s1)

                compute_idx(2 * h + 1, idx_g1, idx_s1, vsc1)
                pltpu.async_copy(pre.at[idx_g1], gb1, sem1)
                pltpu.make_async_copy(pre.at[idx_g0], gb0, sem0).wait()
                scale_scatter(gb0, vsc0, idx_s0, sem_s0)

                @pl.when(h < INNER // 2 - 1)
                def _():
                    wait_scatter(gb0, idx_s0, sem_s0)
                    compute_idx(2 * h + 2, idx_g0, idx_s0, vsc0)
                    pltpu.async_copy(pre.at[idx_g0], gb0, sem0)

                pltpu.make_async_copy(pre.at[idx_g1], gb1, sem1).wait()
                scale_scatter(gb1, vsc1, idx_s1, sem_s1)
                return carry2

            lax.fori_loop(0, INNER // 2, pair_body, 0)
            wait_scatter(gb0, idx_s0, sem_s0)
            wait_scatter(gb1, idx_s1, sem_s1)
            return carry

        lax.fori_loop(0, n_super, super_body, 0)
        plsc.subcore_barrier()
        pltpu.sync_copy(acc.at[pl.ds(sid * ROWS_T, ROWS_T)],
                        out_h.at[cid, pl.ds(sid * ROWS_T, ROWS_T)])
        plsc.subcore_barrier()

    run_phase(nrow, ncol, nval, NSUP_A, True, 0, e_out)
    run_phase(srow, scol, sval, NSUP_B, False, 0, tu_out)
    run_phase(hrow, hcol, hval, NSUP_B, False, NU, ti_out)


_sc_spmm = functools.partial(
    pl.kernel,
    out_type=[
        jax.ShapeDtypeStruct((NC, HP, D), jnp.float32),
        jax.ShapeDtypeStruct((NC, HP, D), jnp.float32),
        jax.ShapeDtypeStruct((NC, HP, D), jnp.float32),
    ],
    mesh=plsc.VectorSubcoreMesh(
        core_axis_name="c", subcore_axis_name="s",
        num_cores=NC, num_subcores=NS),
    compiler_params=pltpu.CompilerParams(use_tc_tiling_on_sc=False),
    scratch_types=[
        pltpu.VMEM((SUP,), jnp.int32),
        pltpu.VMEM((SUP,), jnp.int32),
        pltpu.VMEM((SUP,), jnp.float32),
        pltpu.VMEM((CH,), jnp.int32),
        pltpu.VMEM((CH,), jnp.int32),
        pltpu.VMEM((CH,), jnp.float32),
        pltpu.VMEM((CH,), jnp.int32),
        pltpu.VMEM((CH,), jnp.int32),
        pltpu.VMEM((CH,), jnp.float32),
        pltpu.VMEM((CH, D), jnp.float32),
        pltpu.VMEM((CH, D), jnp.float32),
        pltpu.VMEM((ZR, D), jnp.float32),
        pltpu.VMEM_SHARED((HP, D), jnp.float32),
        pltpu.SemaphoreType.DMA,
        pltpu.SemaphoreType.DMA,
        pltpu.SemaphoreType.DMA,
        pltpu.SemaphoreType.DMA,
    ],
)(_sc_body)


def _pad_edges(idx, val, total, mod):
    e = val.shape[0]
    p = total - e
    ar = jnp.arange(p, dtype=jnp.int32)
    fill = (ar * 7) % mod
    row = jnp.concatenate([idx[0], fill])
    col = jnp.concatenate([idx[1], fill])
    valp = jnp.concatenate([val, jnp.zeros((p,), val.dtype)])
    return row, col, valp


def _user_body(e_ref, u_ref, t0_ref, t1_ref, q_ref, w1_ref, w2_ref, b_ref,
               m_ref, wm_ref, bm_ref, o_ref):
    t = jnp.tanh(e_ref[0] @ q_ref[...])
    g = jnp.tanh(u_ref[...] @ w1_ref[...] + t @ w2_ref[...] + b_ref[...])
    n = jnp.sqrt(jnp.sum(g * g, axis=1, keepdims=True))
    g = g / jnp.maximum(n, 1e-12)
    pair = jnp.tanh((t0_ref[0] + t1_ref[0]) @ m_ref[...])
    ug = g + pair
    o_ref[...] = jnp.tanh(ug @ wm_ref[...] + bm_ref[...])


def _item_body(e_ref, i_ref, t0_ref, t1_ref, q_ref, w1_ref, w2_ref, b_ref,
               m_ref, o_ref):
    t = jnp.tanh(e_ref[0] @ q_ref[...])
    g = jnp.tanh(i_ref[...] @ w1_ref[...] + t @ w2_ref[...] + b_ref[...])
    n = jnp.sqrt(jnp.sum(g * g, axis=1, keepdims=True))
    g = g / jnp.maximum(n, 1e-12)
    o_ref[...] = g + jnp.tanh((t0_ref[0] + t1_ref[0]) @ m_ref[...])


_BT = 1000


def _row_spec(half):
    return pl.BlockSpec((1, _BT, D), lambda i, h=half: (h, i, 0))


def _full(shape):
    return pl.BlockSpec(shape, lambda i: tuple(0 for _ in shape))


def kernel(user_emb, item_emb, norm_idx, norm_val, sym_idx, sym_val,
           herb_idx, herb_val, Q_user, W_gc_user, b_gc_user, Q_item,
           W_gc_item, b_gc_item, M_user, M_item, W_mlp_user, b_mlp_user):
    pre = jnp.concatenate([user_emb, item_emb], axis=0)
    nrow, ncol, nval = _pad_edges(norm_idx, norm_val, EA, NN)
    srow, scol, sval = _pad_edges(sym_idx, sym_val, EB, NU)
    hrow, hcol, hval = _pad_edges(herb_idx, herb_val, EB, NI)

    e2, tup, tip = _sc_spmm(pre, nrow, ncol, nval, srow, scol, sval,
                            hrow, hcol, hval)

    w1u, w2u = W_gc_user[:D], W_gc_user[D:]
    w1i, w2i = W_gc_item[:D], W_gc_item[D:]
    grid = (NU // _BT,)

    out_u = pl.pallas_call(
        _user_body,
        grid=grid,
        in_specs=[
            _row_spec(0),
            pl.BlockSpec((_BT, D), lambda i: (i, 0)),
            _row_spec(0), _row_spec(1),
            _full((D, D)), _full((D, D)), _full((D, D)), _full((1, D)),
            _full((D, D)), _full((D, D)), _full((1, D)),
        ],
        out_specs=pl.BlockSpec((_BT, D), lambda i: (i, 0)),
        out_shape=jax.ShapeDtypeStruct((NU, D), jnp.float32),
    )(e2, user_emb, tup, tup, Q_user, w1u, w2u, b_gc_user,
      M_user, W_mlp_user, b_mlp_user)

    out_i = pl.pallas_call(
        _item_body,
        grid=grid,
        in_specs=[
            _row_spec(1),
            pl.BlockSpec((_BT, D), lambda i: (i, 0)),
            _row_spec(0), _row_spec(1),
            _full((D, D)), _full((D, D)), _full((D, D)), _full((1, D)),
            _full((D, D)),
        ],
        out_specs=pl.BlockSpec((_BT, D), lambda i: (i, 0)),
        out_shape=jax.ShapeDtypeStruct((NI, D), jnp.float32),
    )(e2, item_emb, tip, tip, Q_item, w1i, w2i, b_gc_item, M_item)

    return jnp.concatenate([out_u, out_i], axis=0)


# speedup vs baseline: 6.6676x; 1.0667x over previous
import functools

import jax
import jax.numpy as jnp
from jax import lax
from jax.experimental import pallas as pl
from jax.experimental.pallas import tpu as pltpu
from jax.experimental.pallas import tpu_sc as plsc

NU = 25000
NI = 25000
NN = NU + NI
D = 64
NC = 2
NS = 16
H = 25000
HP = 25088
ROWS_T = HP // NS
CH = 128
INNER = 14
SUP = CH * INNER
NSUP_A = 28
NSUP_B = 7
EA = NS * NSUP_A * SUP
EB = NC * NS * NSUP_B * SUP
ZR = 28
EBUF = SUP + 272


def _sc_body(pre, nrow, ncol, nval, srow, scol, sval, hrow, hcol, hval,
             e_out, tu_out, ti_out,
             rowb, colb, valb, idx_g0, idx_s0, vsc0, idx_g1, idx_s1, vsc1,
             gb0, gb1, zbuf, acc,
             sem0, sem1, sem_s0, sem_s1):
    cid = lax.axis_index("c")
    sid = lax.axis_index("s")
    ii16 = lax.broadcasted_iota(jnp.int32, (16,), 0)
    lanes = [ii16 * 0 + e for e in range(16)]

    def zrow(r, carry):
        for j in range(D // 16):
            zbuf[r, pl.ds(j * 16, 16)] = jnp.zeros((16,), jnp.float32)
        return carry

    lax.fori_loop(0, ZR, zrow, 0)

    def zero_acc():
        zbase = sid * ROWS_T
        for z in range(ROWS_T // ZR):
            pltpu.sync_copy(zbuf, acc.at[pl.ds(zbase + z * ZR, ZR)])
        plsc.subcore_barrier()

    def writeback(out_h):
        plsc.subcore_barrier()
        pltpu.sync_copy(acc.at[pl.ds(sid * ROWS_T, ROWS_T)],
                        out_h.at[cid, pl.ds(sid * ROWS_T, ROWS_T)])
        plsc.subcore_barrier()

    def scale_chunk(gb, vs, voff):
        def scale_body(g, carry3):
            v16 = vs[pl.ds(voff + g * 16, 16)]
            base = g * 16
            for e in range(16):
                bc = v16.at[lanes[e]].get(mode="promise_in_bounds")
                for j in range(D // 16):
                    sl = pl.ds(j * 16, 16)
                    gb[base + e, sl] = gb[base + e, sl] * bc
            return carry3

        lax.fori_loop(0, CH // 16, scale_body, 0)

    def wait_scatter(gb, isc, sem_s):
        pltpu.make_async_copy(gb, acc.at[isc], sem_s).wait()

    def run_phase_a(row_h, col_h, val_h, n_super, out_h):
        zero_acc()
        edge_base = sid * (n_super * SUP)
        lo = cid * H
        shidx = [jnp.maximum(ii16 - k, 0) for k in (1, 2, 4, 8)]
        shmask = [ii16 >= k for k in (1, 2, 4, 8)]

        def prep_idx(isc, ig, off):
            for j in range(CH // 16):
                isc[pl.ds(j * 16, 16)] = rowb[pl.ds(off + j * 16, 16)]
                ig[pl.ds(j * 16, 16)] = colb[pl.ds(off + j * 16, 16)]

        def super_body(g, carry):
            sb = edge_base + g * SUP
            pltpu.sync_copy(row_h.at[pl.ds(sb, EBUF)], rowb)
            pltpu.sync_copy(col_h.at[pl.ds(sb, EBUF)], colb)
            pltpu.sync_copy(val_h.at[pl.ds(sb, EBUF)], valb)

            def fbody(j, p):
                slb = pl.ds(j * 16, 16)
                r = rowb[slb]
                cc = colb[slb]
                vv = valb[slb]
                m = r - jnp.where(r >= H, H, 0)
                ok = (r >= lo) & (r < lo + H)
                s = jnp.where(ok, 1, 0)
                for t in range(4):
                    sh = s.at[shidx[t]].get(mode="promise_in_bounds")
                    s = s + jnp.where(shmask[t], sh, 0)
                sel = ii16 * 0
                for step in (8, 4, 2, 1):
                    sv = s.at[sel + (step - 1)].get(mode="promise_in_bounds")
                    sel = jnp.where(sv < ii16 + 1, sel + step, sel)
                slw = pl.ds(p, 16)
                rowb[slw] = m.at[sel].get(mode="promise_in_bounds")
                colb[slw] = cc.at[sel].get(mode="promise_in_bounds")
                valb[slw] = vv.at[sel].get(mode="promise_in_bounds")
                return p + s[15]

            p = lax.fori_loop(0, SUP // 16, fbody, 0)
            for j in range(16):
                slp = pl.ds(p + j * 16, 16)
                rowb[slp] = ii16 + (j * 16)
                colb[slp] = ii16 + (j * 16)
                valb[slp] = jnp.zeros((16,), jnp.float32)
            npair = (p + 255) // 256

            @pl.when(npair > 0)
            def _():
                prep_idx(idx_s0, idx_g0, 0)
                pltpu.async_copy(pre.at[idx_g0], gb0, sem0)

            def pair_body(i, carry2):
                c0 = i * 256
                c1 = c0 + CH

                @pl.when(i > 0)
                def _():
                    wait_scatter(gb1, idx_s1, sem_s1)

                prep_idx(idx_s1, idx_g1, c1)
                pltpu.async_copy(pre.at[idx_g1], gb1, sem1)
                pltpu.make_async_copy(pre.at[idx_g0], gb0, sem0).wait()
                scale_chunk(gb0, valb, c0)
                pltpu.async_copy(gb0, acc.at[idx_s0], sem_s0, add=True)

                @pl.when(i < npair - 1)
                def _():
                    wait_scatter(gb0, idx_s0, sem_s0)
                    prep_idx(idx_s0, idx_g0, c0 + 256)
                    pltpu.async_copy(pre.at[idx_g0], gb0, sem0)

                pltpu.make_async_copy(pre.at[idx_g1], gb1, sem1).wait()
                scale_chunk(gb1, valb, c1)
                pltpu.async_copy(gb1, acc.at[idx_s1], sem_s1, add=True)
                return carry2

            lax.fori_loop(0, npair, pair_body, 0)

            @pl.when(npair > 0)
            def _():
                wait_scatter(gb0, idx_s0, sem_s0)
                wait_scatter(gb1, idx_s1, sem_s1)

            return carry

        lax.fori_loop(0, n_super, super_body, 0)
        writeback(out_h)

    def run_phase(row_h, col_h, val_h, n_super, col_off, out_h):
        zero_acc()
        edge_base = (cid * NS + sid) * (n_super * SUP)

        def compute_idx(ci, ig, isc, vs):
            cb = ci * CH
            for j in range(CH // 16):
                sl16 = pl.ds(j * 16, 16)
                slb = pl.ds(cb + j * 16, 16)
                isc[sl16] = rowb[slb]
                ig[sl16] = colb[slb] + col_off
                vs[sl16] = valb[slb]

        def scale_scatter(gb, vs, isc, sem_s):
            scale_chunk(gb, vs, 0)
            pltpu.async_copy(gb, acc.at[isc], sem_s, add=True)

        def super_body(g, carry):
            sb = edge_base + g * SUP
            pltpu.sync_copy(row_h.at[pl.ds(sb, EBUF)], rowb)
            pltpu.sync_copy(col_h.at[pl.ds(sb, EBUF)], colb)
            pltpu.sync_copy(val_h.at[pl.ds(sb, EBUF)], valb)

            compute_idx(0, idx_g0, idx_s0, vsc0)
            pltpu.async_copy(pre.at[idx_g0], gb0, sem0)

            def pair_body(h, carry2):
                @pl.when(h > 0)
                def _():
                    wait_scatter(gb1, idx_s1, sem_s1)

                compute_idx(2 * h + 1, idx_g1, idx_s1, vsc1)
                pltpu.async_copy(pre.at[idx_g1], gb1, sem1)
                pltpu.make_async_copy(pre.at[idx_g0], gb0, sem0).wait()
                scale_scatter(gb0, vsc0, idx_s0, sem_s0)

                @pl.when(h < INNER // 2 - 1)
                def _():
                    wait_scatter(gb0, idx_s0, sem_s0)
                    compute_idx(2 * h + 2, idx_g0, idx_s0, vsc0)
                    pltpu.async_copy(pre.at[idx_g0], gb0, sem0)

                pltpu.make_async_copy(pre.at[idx_g1], gb1, sem1).wait()
                scale_scatter(gb1, vsc1, idx_s1, sem_s1)
                return carry2

            lax.fori_loop(0, INNER // 2, pair_body, 0)
            wait_scatter(gb0, idx_s0, sem_s0)
            wait_scatter(gb1, idx_s1, sem_s1)
            return carry

        lax.fori_loop(0, n_super, super_body, 0)
        writeback(out_h)

    run_phase_a(nrow, ncol, nval, NSUP_A, e_out)
    run_phase(srow, scol, sval, NSUP_B, 0, tu_out)
    run_phase(hrow, hcol, hval, NSUP_B, NU, ti_out)


_sc_spmm = functools.partial(
    pl.kernel,
    out_type=[
        jax.ShapeDtypeStruct((NC, HP, D), jnp.float32),
        jax.ShapeDtypeStruct((NC, HP, D), jnp.float32),
        jax.ShapeDtypeStruct((NC, HP, D), jnp.float32),
    ],
    mesh=plsc.VectorSubcoreMesh(
        core_axis_name="c", subcore_axis_name="s",
        num_cores=NC, num_subcores=NS),
    compiler_params=pltpu.CompilerParams(use_tc_tiling_on_sc=False),
    scratch_types=[
        pltpu.VMEM((EBUF,), jnp.int32),
        pltpu.VMEM((EBUF,), jnp.int32),
        pltpu.VMEM((EBUF,), jnp.float32),
        pltpu.VMEM((CH,), jnp.int32),
        pltpu.VMEM((CH,), jnp.int32),
        pltpu.VMEM((CH,), jnp.float32),
        pltpu.VMEM((CH,), jnp.int32),
        pltpu.VMEM((CH,), jnp.int32),
        pltpu.VMEM((CH,), jnp.float32),
        pltpu.VMEM((CH, D), jnp.float32),
        pltpu.VMEM((CH, D), jnp.float32),
        pltpu.VMEM((ZR, D), jnp.float32),
        pltpu.VMEM_SHARED((HP, D), jnp.float32),
        pltpu.SemaphoreType.DMA,
        pltpu.SemaphoreType.DMA,
        pltpu.SemaphoreType.DMA,
        pltpu.SemaphoreType.DMA,
    ],
)(_sc_body)


def _pad_edges(idx, val, total, mod):
    e = val.shape[0]
    p = total - e
    ar = jnp.arange(p, dtype=jnp.int32)
    fill = (ar * 7) % mod
    row = jnp.concatenate([idx[0], fill])
    col = jnp.concatenate([idx[1], fill])
    valp = jnp.concatenate([val, jnp.zeros((p,), val.dtype)])
    return row, col, valp


def _user_body(e_ref, u_ref, t0_ref, t1_ref, q_ref, w1_ref, w2_ref, b_ref,
               m_ref, wm_ref, bm_ref, o_ref):
    t = jnp.tanh(e_ref[0] @ q_ref[...])
    g = jnp.tanh(u_ref[...] @ w1_ref[...] + t @ w2_ref[...] + b_ref[...])
    n = jnp.sqrt(jnp.sum(g * g, axis=1, keepdims=True))
    g = g / jnp.maximum(n, 1e-12)
    pair = jnp.tanh((t0_ref[0] + t1_ref[0]) @ m_ref[...])
    ug = g + pair
    o_ref[...] = jnp.tanh(ug @ wm_ref[...] + bm_ref[...])


def _item_body(e_ref, i_ref, t0_ref, t1_ref, q_ref, w1_ref, w2_ref, b_ref,
               m_ref, o_ref):
    t = jnp.tanh(e_ref[0] @ q_ref[...])
    g = jnp.tanh(i_ref[...] @ w1_ref[...] + t @ w2_ref[...] + b_ref[...])
    n = jnp.sqrt(jnp.sum(g * g, axis=1, keepdims=True))
    g = g / jnp.maximum(n, 1e-12)
    o_ref[...] = g + jnp.tanh((t0_ref[0] + t1_ref[0]) @ m_ref[...])


_BT = 1000


def _row_spec(half):
    return pl.BlockSpec((1, _BT, D), lambda i, h=half: (h, i, 0))


def _full(shape):
    return pl.BlockSpec(shape, lambda i: tuple(0 for _ in shape))


def kernel(user_emb, item_emb, norm_idx, norm_val, sym_idx, sym_val,
           herb_idx, herb_val, Q_user, W_gc_user, b_gc_user, Q_item,
           W_gc_item, b_gc_item, M_user, M_item, W_mlp_user, b_mlp_user):
    pre = jnp.concatenate([user_emb, item_emb], axis=0)
    nrow, ncol, nval = _pad_edges(norm_idx, norm_val, EA + 272, NN)
    srow, scol, sval = _pad_edges(sym_idx, sym_val, EB + 272, NU)
    hrow, hcol, hval = _pad_edges(herb_idx, herb_val, EB + 272, NI)

    e2, tup, tip = _sc_spmm(pre, nrow, ncol, nval, srow, scol, sval,
                            hrow, hcol, hval)

    w1u, w2u = W_gc_user[:D], W_gc_user[D:]
    w1i, w2i = W_gc_item[:D], W_gc_item[D:]
    grid = (NU // _BT,)

    out_u = pl.pallas_call(
        _user_body,
        grid=grid,
        in_specs=[
            _row_spec(0),
            pl.BlockSpec((_BT, D), lambda i: (i, 0)),
            _row_spec(0), _row_spec(1),
            _full((D, D)), _full((D, D)), _full((D, D)), _full((1, D)),
            _full((D, D)), _full((D, D)), _full((1, D)),
        ],
        out_specs=pl.BlockSpec((_BT, D), lambda i: (i, 0)),
        out_shape=jax.ShapeDtypeStruct((NU, D), jnp.float32),
    )(e2, user_emb, tup, tup, Q_user, w1u, w2u, b_gc_user,
      M_user, W_mlp_user, b_mlp_user)

    out_i = pl.pallas_call(
        _item_body,
        grid=grid,
        in_specs=[
            _row_spec(1),
            pl.BlockSpec((_BT, D), lambda i: (i, 0)),
            _row_spec(0), _row_spec(1),
            _full((D, D)), _full((D, D)), _full((D, D)), _full((1, D)),
            _full((D, D)),
        ],
        out_specs=pl.BlockSpec((_BT, D), lambda i: (i, 0)),
        out_shape=jax.ShapeDtypeStruct((NI, D), jnp.float32),
    )(e2, item_emb, tip, tip, Q_item, w1i, w2i, b_gc_item, M_item)

    return jnp.concatenate([out_u, out_i], axis=0)
